# Initial kernel scaffold; baseline (speedup 1.0000x reference)
import jax
import jax.numpy as jnp
from jax.experimental import pallas as pl


def kernel(x, edge_index, W_gcn, b_gcn, gamma, beta, W_score, b_score, W0, b0, W1, b1, W2, b2):
    raise NotImplementedError("write your pallas kernel here")


# speedup vs baseline: 17.7806x; 17.7806x over previous
import functools

import jax
import jax.numpy as jnp
from jax import lax
from jax.experimental import pallas as pl
from jax.experimental.pallas import tpu as pltpu
from jax.experimental.pallas import tpu_sc as plsc

_N = 10000
_E = 320000
_D = 128
_K = 5000
_NC, _NS = 2, 16

_AROWS = _E // 128
_CROWS = 2560
_RPT = _CROWS // (_NC * _NS)
_NPAD = 10240
_AGGR = 10304
_ZSTR = _AGGR // _NS
_OSTR = _NPAD // _NS

_INT_MIN_PY = -2147483648

_sc_mesh = functools.partial(
    plsc.VectorSubcoreMesh, core_axis_name="c", subcore_axis_name="s",
    num_cores=_NC, num_subcores=_NS)


def _load_worker_rows(w, src_hbm, dst_hbm, srcb, dstb):
    @pl.when(w < 24)
    def _():
        start = 80 * w
        pltpu.sync_copy(src_hbm.at[pl.ds(start, 80)], srcb)
        pltpu.sync_copy(dst_hbm.at[pl.ds(start, 80)], dstb)

    @pl.when(w >= 24)
    def _():
        start = 1920 + 72 * (w - 24)
        pltpu.sync_copy(src_hbm.at[pl.ds(start, 72)], srcb.at[pl.ds(0, 72)])
        pltpu.sync_copy(dst_hbm.at[pl.ds(start, 72)], dstb.at[pl.ds(0, 72)])

    @pl.when(w == 31)
    def _():
        pltpu.sync_copy(src_hbm.at[pl.ds(2496, 4)], srcb.at[pl.ds(72, 4)])
        pltpu.sync_copy(dst_hbm.at[pl.ds(2496, 4)], dstb.at[pl.ds(72, 4)])

    return jnp.where(w < 24, 80, jnp.where(w == 31, 76, 72))


def _deg_body(src_hbm, dst_hbm, zero_hbm, dout_hbm, din_hbm,
              srcb, dstb, onesb, dout_sh, din_sh, sem):
    c = lax.axis_index("c")
    s = lax.axis_index("s")
    w = c * _NS + s
    pltpu.sync_copy(zero_hbm.at[pl.ds(s * _OSTR, _OSTR)],
                    dout_sh.at[pl.ds(s * _OSTR, _OSTR)])
    pltpu.sync_copy(zero_hbm.at[pl.ds(s * _OSTR, _OSTR)],
                    din_sh.at[pl.ds(s * _OSTR, _OSTR)])
    for k in range(8):
        onesb[pl.ds(16 * k, 16)] = jnp.full((16,), 1.0, jnp.float32)
    cnt = _load_worker_rows(w, src_hbm, dst_hbm, srcb, dstb)
    plsc.subcore_barrier()

    def body(r, _):
        pltpu.async_copy(onesb, dout_sh.at[srcb.at[r]], sem, add=True)
        pltpu.async_copy(onesb, din_sh.at[dstb.at[r]], sem, add=True)
        return 0
    lax.fori_loop(0, cnt, body, 0)

    def drain(r, _):
        pltpu.make_async_copy(onesb, dout_sh.at[srcb.at[0]], sem).wait()
        return 0
    lax.fori_loop(0, 2 * cnt, drain, 0)
    plsc.subcore_barrier()
    pltpu.sync_copy(dout_sh.at[pl.ds(s * _OSTR, _OSTR)],
                    dout_hbm.at[pl.ds(c * _NPAD + s * _OSTR, _OSTR)])
    pltpu.sync_copy(din_sh.at[pl.ds(s * _OSTR, _OSTR)],
                    din_hbm.at[pl.ds(c * _NPAD + s * _OSTR, _OSTR)])


@functools.cache
def _deg_call():
    return pl.kernel(
        _deg_body,
        out_type=[jax.ShapeDtypeStruct((_NC * _NPAD,), jnp.float32),
                  jax.ShapeDtypeStruct((_NC * _NPAD,), jnp.float32)],
        mesh=_sc_mesh(),
        scratch_types=[
            pltpu.VMEM((80, 128), jnp.int32),
            pltpu.VMEM((80, 128), jnp.int32),
            pltpu.VMEM((128,), jnp.float32),
            pltpu.VMEM_SHARED((_NPAD,), jnp.float32),
            pltpu.VMEM_SHARED((_NPAD,), jnp.float32),
            pltpu.SemaphoreType.DMA,
        ])


def _conv_body(src_hbm, dst_hbm, h_hbm, zero_hbm, agg0_hbm, agg1_hbm,
               srcb, dstb, msgA, msgB, agg_sh, semA, semB):
    c = lax.axis_index("c")
    s = lax.axis_index("s")
    base = (c * _NS + s) * _RPT
    pltpu.sync_copy(zero_hbm.at[pl.ds(s * 640, 640)],
                    agg_sh.at[pl.ds(s * 640, 640)])

    @pl.when(s == 0)
    def _():
        pltpu.sync_copy(zero_hbm.at[pl.ds(10240, 64)],
                        agg_sh.at[pl.ds(10240, 64)])

    plsc.subcore_barrier()

    def blk(bi, _):
        b0 = base + 16 * bi
        pltpu.sync_copy(src_hbm.at[pl.ds(b0, 16)], srcb)
        pltpu.sync_copy(dst_hbm.at[pl.ds(b0, 16)], dstb)
        pltpu.async_copy(h_hbm.at[srcb.at[0]], msgA, semA)

        def pair(i, _):
            j = 2 * i
            pltpu.async_copy(h_hbm.at[srcb.at[j + 1]], msgB, semB)
            pltpu.make_async_copy(h_hbm.at[srcb.at[0]], msgA, semA).wait()
            pltpu.sync_copy(msgA, agg_sh.at[dstb.at[j]], add=True)

            @pl.when(j + 2 < 16)
            def _():
                pltpu.async_copy(h_hbm.at[srcb.at[j + 2]], msgA, semA)

            pltpu.make_async_copy(h_hbm.at[srcb.at[0]], msgB, semB).wait()
            pltpu.sync_copy(msgB, agg_sh.at[dstb.at[j + 1]], add=True)
            return 0
        lax.fori_loop(0, 8, pair, 0)
        return 0
    lax.fori_loop(0, _RPT // 16, blk, 0)

    plsc.subcore_barrier()

    @pl.when(c == 0)
    def _():
        pltpu.sync_copy(agg_sh.at[pl.ds(s * _OSTR, _OSTR)],
                        agg0_hbm.at[pl.ds(s * _OSTR, _OSTR)])

    @pl.when(c == 1)
    def _():
        pltpu.sync_copy(agg_sh.at[pl.ds(s * _OSTR, _OSTR)],
                        agg1_hbm.at[pl.ds(s * _OSTR, _OSTR)])


@functools.cache
def _conv_call():
    return pl.kernel(
        _conv_body,
        out_type=[jax.ShapeDtypeStruct((_NPAD, _D), jnp.float32),
                  jax.ShapeDtypeStruct((_NPAD, _D), jnp.float32)],
        mesh=_sc_mesh(),
        scratch_types=[
            pltpu.VMEM((16, 128), jnp.int32),
            pltpu.VMEM((16, 128), jnp.int32),
            pltpu.VMEM((128, _D), jnp.float32),
            pltpu.VMEM((128, _D), jnp.float32),
            pltpu.VMEM_SHARED((_AGGR, _D), jnp.float32),
            pltpu.SemaphoreType.DMA,
            pltpu.SemaphoreType.DMA,
        ])


def _score_body(src_hbm, dst_hbm, s1n_hbm, zero_hbm, sagg_hbm,
                srcb, dstb, valb, sagg_sh, semg, sems):
    c = lax.axis_index("c")
    s = lax.axis_index("s")
    w = c * _NS + s
    pltpu.sync_copy(zero_hbm.at[pl.ds(s * _OSTR, _OSTR)],
                    sagg_sh.at[pl.ds(s * _OSTR, _OSTR)])
    cnt = _load_worker_rows(w, src_hbm, dst_hbm, srcb, dstb)
    plsc.subcore_barrier()

    pltpu.async_copy(s1n_hbm.at[srcb.at[0]], valb.at[0], semg)

    def body(r, _):
        @pl.when(r + 1 < cnt)
        def _():
            pltpu.async_copy(s1n_hbm.at[srcb.at[r + 1]], valb.at[r + 1], semg)
        pltpu.make_async_copy(s1n_hbm.at[srcb.at[0]], valb.at[0], semg).wait()
        pltpu.async_copy(valb.at[r], sagg_sh.at[dstb.at[r]], sems, add=True)
        return 0
    lax.fori_loop(0, cnt, body, 0)

    def drain(r, _):
        pltpu.make_async_copy(valb.at[0], sagg_sh.at[dstb.at[0]], sems).wait()
        return 0
    lax.fori_loop(0, cnt, drain, 0)
    plsc.subcore_barrier()
    pltpu.sync_copy(sagg_sh.at[pl.ds(s * _OSTR, _OSTR)],
                    sagg_hbm.at[pl.ds(c * _NPAD + s * _OSTR, _OSTR)])


@functools.cache
def _score_call():
    return pl.kernel(
        _score_body,
        out_type=jax.ShapeDtypeStruct((_NC * _NPAD,), jnp.float32),
        mesh=_sc_mesh(),
        scratch_types=[
            pltpu.VMEM((80, 128), jnp.int32),
            pltpu.VMEM((80, 128), jnp.int32),
            pltpu.VMEM((80, 128), jnp.float32),
            pltpu.VMEM_SHARED((_NPAD,), jnp.float32),
            pltpu.SemaphoreType.DMA,
            pltpu.SemaphoreType.DMA,
        ])


def _h1n_body(x_ref, w_ref, d0_ref, d1_ref, out_ref):
    deg = d0_ref[...] + d1_ref[...]
    nsrc = jnp.where(deg > 0, lax.rsqrt(jnp.maximum(deg, 1.0)), 0.0)
    h1 = jnp.dot(x_ref[...], w_ref[...], preferred_element_type=jnp.float32)
    out_ref[...] = h1 * nsrc


def _hsn_body(agg0, agg1, x_ref, di0, di1, do0, do1, bg, gm, bt, ws,
              h_out, s1n_out):
    a = agg0[pl.ds(0, _N), :] + agg1[pl.ds(0, _N), :]
    degi = di0[...] + di1[...]
    ndst = jnp.where(degi > 0, lax.rsqrt(jnp.maximum(degi, 1.0)), 0.0)
    a = a * ndst + bg[...]
    mean = jnp.sum(a, axis=0, keepdims=True) * (1.0 / _N)
    d = a - mean
    var = jnp.sum(d * d, axis=0, keepdims=True) * (1.0 / _N)
    hn = d * lax.rsqrt(var + 1e-5) * gm[...] + bt[...]
    h = x_ref[...] + jnp.maximum(hn, 0.0)
    h_out[...] = h
    dego = do0[...] + do1[...]
    nsrc = jnp.where(dego > 0, lax.rsqrt(jnp.maximum(dego, 1.0)), 0.0)
    s1 = jnp.dot(h, ws[...], preferred_element_type=jnp.float32)
    s1n_out[...] = s1 * nsrc


def _sel_body(sg0, sg1, di0, di1, bs, wm_out, wt_out, m_out):
    degi = di0[...] + di1[...]
    ndst = jnp.where(degi > 0, lax.rsqrt(jnp.maximum(degi, 1.0)), 0.0)
    score = (sg0[...] + sg1[...]) * ndst + bs[0, 0]
    rid = lax.broadcasted_iota(jnp.int32, (80, 128), 0)
    cid = lax.broadcasted_iota(jnp.int32, (80, 128), 1)
    nid = rid * 128 + cid
    valid = nid < _N
    int_min = jnp.int32(_INT_MIN_PY)
    si = lax.bitcast_convert_type(score, jnp.int32)
    mkey = jnp.where(si >= 0, si, si ^ jnp.int32(0x7FFFFFFF))
    mkey = jnp.where(valid, mkey, int_min)

    def bit_body(i, p):
        b = 31 - i
        cand = p | (jnp.int32(1) << b)
        cand_s = cand ^ int_min
        cnt = jnp.sum(jnp.where(mkey >= cand_s, 1, 0))
        return jnp.where(cnt >= _K, cand, p)
    p = lax.fori_loop(0, 32, bit_body, jnp.int32(0))
    v_s = p ^ int_min
    c_gt = jnp.sum(jnp.where(mkey > v_s, 1, 0))
    need = _K - c_gt
    tie = mkey == v_s

    def idx_body(i, xacc):
        b = 13 - i
        t = xacc + (jnp.int32(1) << b) - 1
        cnt = jnp.sum(jnp.where(tie & (nid <= t), 1, 0))
        return jnp.where(cnt >= need, xacc, xacc + (jnp.int32(1) << b))
    xthr = lax.fori_loop(0, 14, idx_body, jnp.int32(0))
    sel = (mkey > v_s) | (tie & (nid <= xthr))
    th = jnp.tanh(score)
    wt = jnp.where(sel, th, 0.0)
    wt_out[...] = wt
    wm_out[...] = wt * (1.0 / _K)
    m_out[...] = jnp.where(sel, 1.0, 0.0)


def _readout_body(h_ref, wm, wt, m, w0, b0_, w1, b1_, w2, b2_, y_out):
    hv = h_ref[...]
    mean = lax.dot_general(wm[...], hv, (((0,), (0,)), ((), ())),
                           preferred_element_type=jnp.float32)
    neg = jnp.float32(-3.0e38)
    mx = jnp.max(jnp.where(m[...] > 0, hv * wt[...], neg),
                 axis=0, keepdims=True)
    hg = jnp.concatenate([mean, mx], axis=1)
    y = jnp.dot(hg, w0[...], preferred_element_type=jnp.float32) + b0_[...]
    y = jnp.maximum(y, 0.0)
    y = jnp.dot(y, w1[...], preferred_element_type=jnp.float32) + b1_[...]
    y = jnp.maximum(y, 0.0)
    y_out[...] = jnp.dot(y, w2[...], preferred_element_type=jnp.float32) + b2_[...]


def _sds(shape):
    return jax.ShapeDtypeStruct(shape, jnp.float32)


def kernel(x, edge_index, W_gcn, b_gcn, gamma, beta, W_score, b_score,
           W0, b0, W1, b1, W2, b2):
    src = edge_index[0]
    dst = edge_index[1]
    src2d = src.reshape(_AROWS, 128)
    dst2d = dst.reshape(_AROWS, 128)
    padn = _CROWS * 128 - _E
    pad_src = (jnp.arange(padn, dtype=jnp.int32) % _N)
    pad_dst = _NPAD + (jnp.arange(padn, dtype=jnp.int32) % 64)
    src_p = jnp.concatenate([src, pad_src]).reshape(_CROWS, 128)
    dst_p = jnp.concatenate([dst, pad_dst]).reshape(_CROWS, 128)
    z1 = jnp.zeros((_NPAD,), jnp.float32)
    z2 = jnp.zeros((_AGGR, _D), jnp.float32)

    dout_f, din_f = _deg_call()(src2d, dst2d, z1)
    dout_p = dout_f.reshape(_NC, _NPAD)
    din_p = din_f.reshape(_NC, _NPAD)
    col = lambda a: a.reshape(-1, 1)[:_N]
    do0, do1 = col(dout_p[0]), col(dout_p[1])
    di0, di1 = col(din_p[0]), col(din_p[1])

    h1n = pl.pallas_call(
        _h1n_body, out_shape=_sds((_N, _D)),
    )(x, W_gcn, do0, do1)

    agg0, agg1 = _conv_call()(src_p, dst_p, h1n, z2)

    h, s1n = pl.pallas_call(
        _hsn_body, out_shape=[_sds((_N, _D)), _sds((_N, 1))],
    )(agg0, agg1, x, di0, di1, do0, do1, b_gcn.reshape(1, -1),
      gamma.reshape(1, -1), beta.reshape(1, -1), W_score)

    sagg = _score_call()(src2d, dst2d, s1n.reshape(-1), z1).reshape(_NC, _NPAD)

    lane = lambda a: a.reshape(80, 128)
    wm, wt, m = pl.pallas_call(
        _sel_body, out_shape=[_sds((80, 128))] * 3,
    )(lane(sagg[0]), lane(sagg[1]), lane(din_p[0]), lane(din_p[1]),
      b_score.reshape(1, 1))

    y = pl.pallas_call(
        _readout_body, out_shape=_sds((1, 10)),
    )(h, col(wm), col(wt), col(m), W0, b0.reshape(1, -1), W1,
      b1.reshape(1, -1), W2, b2.reshape(1, -1))
    return y
